# Initial kernel scaffold; baseline (speedup 1.0000x reference)
import jax
import jax.numpy as jnp
from jax.experimental import pallas as pl


def kernel(x, Wq, bq, Wk, bk, Wv, bv, Wo, bo):
    raise NotImplementedError("write your pallas kernel here")


# speedup vs baseline: 8.3386x; 8.3386x over previous
import functools

import jax
import jax.numpy as jnp
from jax import lax
from jax.experimental import pallas as pl
from jax.experimental.pallas import tpu as pltpu
from jax.experimental.pallas import tpu_sc as plsc

B, C, H, W, NREF = 2, 192, 224, 224, 4
HW = H * W
BHW = B * HW
CP = 256


BLK = 1024
N_BLK = HW // BLK


def _proj_body(x_ref, wq_ref, bq_ref, wk_ref, bk_ref, wv_ref, bv_ref,
               wo_ref, bo_ref, qt_ref, kt_ref, vt_ref, idx_ref):
    b = pl.program_id(0)
    j = pl.program_id(1)
    xb = x_ref[0]
    cdims = (((0,), (1,)), ((), ()))
    pad = jnp.zeros((BLK, CP - C), jnp.float32)
    qt = lax.dot_general(xb, wq_ref[...], cdims,
                         preferred_element_type=jnp.float32) + bq_ref[...]
    qt_ref[...] = qt
    kt = lax.dot_general(xb, wk_ref[...], cdims,
                         preferred_element_type=jnp.float32) + bk_ref[...]
    kt_ref[...] = jnp.concatenate([kt, pad], axis=1)
    vt = lax.dot_general(xb, wv_ref[...], cdims,
                         preferred_element_type=jnp.float32) + bv_ref[...]
    vt_ref[...] = jnp.concatenate([vt, pad], axis=1)
    offs = lax.dot_general(wo_ref[...], qt, (((1,), (1,)), ((), ())),
                           preferred_element_type=jnp.float32) + bo_ref[...]
    p = j * BLK + lax.broadcasted_iota(jnp.int32, (1, BLK), 1)
    hpos = (p // W).astype(jnp.float32)
    wpos = (p % W).astype(jnp.float32)
    offs = offs.reshape(NREF, 2, BLK)
    ref_w = jnp.clip(wpos + offs[:, 0, :], 0.0, float(W - 1)).astype(jnp.int32)
    ref_h = jnp.clip(hpos + offs[:, 1, :], 0.0, float(H - 1)).astype(jnp.int32)
    idx_ref[...] = ref_h * W + ref_w + b * HW


def _stage1(x, Wq, bq, Wk, bk, Wv, bv, Wo, bo):
    xf = x.reshape(B, C, HW)
    grid = (B, N_BLK)
    wspec = pl.BlockSpec((C, C), lambda b, j: (0, 0))
    rspec = pl.BlockSpec((1, C), lambda b, j: (0, 0))
    return pl.pallas_call(
        _proj_body,
        grid=grid,
        in_specs=[
            pl.BlockSpec((1, C, BLK), lambda b, j: (b, 0, j)),
            wspec, rspec,
            wspec, rspec,
            wspec, rspec,
            pl.BlockSpec((2 * NREF, C), lambda b, j: (0, 0)),
            pl.BlockSpec((2 * NREF, 1), lambda b, j: (0, 0)),
        ],
        out_specs=[
            pl.BlockSpec((BLK, C), lambda b, j: (b * N_BLK + j, 0)),
            pl.BlockSpec((BLK, CP), lambda b, j: (b * N_BLK + j, 0)),
            pl.BlockSpec((BLK, CP), lambda b, j: (b * N_BLK + j, 0)),
            pl.BlockSpec((NREF, BLK), lambda b, j: (0, b * N_BLK + j)),
        ],
        out_shape=[
            jax.ShapeDtypeStruct((BHW, C), jnp.float32),
            jax.ShapeDtypeStruct((BHW, CP), jnp.float32),
            jax.ShapeDtypeStruct((BHW, CP), jnp.float32),
            jax.ShapeDtypeStruct((NREF, BHW), jnp.int32),
        ],
    )(xf, Wq, bq.reshape(1, C), Wk, bk.reshape(1, C), Wv, bv.reshape(1, C),
      Wo, bo.reshape(2 * NREF, 1))


NC, NS, L = 2, 16, 16
NW = NC * NS
P = 128
SUB = 16
NSUB = P // SUB
NCHUNK = BHW // P
NJ = C // L


def _sc_body(qt_hbm, kt_hbm, vt_hbm, idx_hbm, out_hbm,
             idxv, kg, vg, qv, ov, sem):
    wid = lax.axis_index("s") * NC + lax.axis_index("c")
    nchunks = jnp.where(wid < NCHUNK % NW, NCHUNK // NW + 1, NCHUNK // NW)

    def chunk_body(t, _):
        ci = wid + t * NW
        base = pl.multiple_of(ci * P, P)
        pltpu.sync_copy(idx_hbm.at[:, pl.ds(base, P)], idxv)
        pltpu.sync_copy(qt_hbm.at[pl.ds(base, P)], qv)
        for sub in range(NSUB):
            copies = []
            for r in range(NREF):
                ixr = idxv.at[r, pl.ds(sub * SUB, SUB)]
                copies.append(pltpu.async_copy(kt_hbm.at[ixr], kg.at[r], sem))
                copies.append(pltpu.async_copy(vt_hbm.at[ixr], vg.at[r], sem))
            for cp in copies:
                cp.wait()

            @plsc.parallel_loop(0, SUB, 1, unroll=2)
            def pos_body(i):
                pi = sub * SUB + i
                qvecs = []
                for jgrp in range(NJ):
                    qvecs.append(qv[pi, pl.ds(jgrp * L, L)])
                s = []
                for r in range(NREF):
                    acc = qvecs[0] * kg[r, i, pl.ds(0, L)]
                    for jgrp in range(1, NJ):
                        acc = acc + qvecs[jgrp] * kg[r, i, pl.ds(jgrp * L, L)]
                    s.append(jnp.sum(acc))
                for jgrp in range(NJ):
                    o = s[0] * vg[0, i, pl.ds(jgrp * L, L)]
                    for r in range(1, NREF):
                        o = o + s[r] * vg[r, i, pl.ds(jgrp * L, L)]
                    ov[pi, pl.ds(jgrp * L, L)] = o

        pltpu.sync_copy(ov, out_hbm.at[pl.ds(base, P)])
        return ()

    lax.fori_loop(0, nchunks, chunk_body, (), unroll=False)


@functools.cache
def _sc_attn():
    return pl.kernel(
        _sc_body,
        out_type=jax.ShapeDtypeStruct((BHW, C), jnp.float32),
        mesh=plsc.VectorSubcoreMesh(core_axis_name="c", subcore_axis_name="s",
                                    num_cores=NC, num_subcores=NS),
        compiler_params=pltpu.CompilerParams(needs_layout_passes=False),
        scratch_types=[
            pltpu.VMEM((NREF, P), jnp.int32),
            pltpu.VMEM((NREF, SUB, CP), jnp.float32),
            pltpu.VMEM((NREF, SUB, CP), jnp.float32),
            pltpu.VMEM((P, C), jnp.float32),
            pltpu.VMEM((P, C), jnp.float32),
            pltpu.SemaphoreType.DMA,
        ],
    )


TBLK = 512
NT_BLK = HW // TBLK


def _tr_body(ot_ref, o_ref):
    o_ref[0] = ot_ref[...].T


def _stage3(outT):
    return pl.pallas_call(
        _tr_body,
        grid=(B, NT_BLK),
        in_specs=[pl.BlockSpec((TBLK, C), lambda b, j: (b * NT_BLK + j, 0))],
        out_specs=pl.BlockSpec((1, C, TBLK), lambda b, j: (b, 0, j)),
        out_shape=jax.ShapeDtypeStruct((B, C, HW), jnp.float32),
    )(outT)


def kernel(x, Wq, bq, Wk, bk, Wv, bv, Wo, bo):
    qt, kt, vt, idx = _stage1(x, Wq, bq, Wk, bk, Wv, bv, Wo, bo)
    outT = _sc_attn()(qt, kt, vt, idx)
    return _stage3(outT).reshape(B, C, H, W)


# speedup vs baseline: 10.2996x; 1.2352x over previous
import functools

import jax
import jax.numpy as jnp
from jax import lax
from jax.experimental import pallas as pl
from jax.experimental.pallas import tpu as pltpu
from jax.experimental.pallas import tpu_sc as plsc

B, C, H, W, NREF = 2, 192, 224, 224, 4
HW = H * W
BHW = B * HW
CP = 256


BLK = 1024
N_BLK = HW // BLK


def _proj_body(x_ref, wq_ref, bq_ref, wk_ref, bk_ref, wv_ref, bv_ref,
               wo_ref, bo_ref, qt_ref, kt_ref, vt_ref, idx_ref):
    b = pl.program_id(0)
    j = pl.program_id(1)
    xb = x_ref[0]
    cdims = (((0,), (1,)), ((), ()))
    pad = jnp.zeros((BLK, CP - C), jnp.float32)
    qt = lax.dot_general(xb, wq_ref[...], cdims,
                         preferred_element_type=jnp.float32) + bq_ref[...]
    qt_ref[...] = qt
    kt = lax.dot_general(xb, wk_ref[...], cdims,
                         preferred_element_type=jnp.float32) + bk_ref[...]
    kt_ref[...] = jnp.concatenate([kt, pad], axis=1)
    vt = lax.dot_general(xb, wv_ref[...], cdims,
                         preferred_element_type=jnp.float32) + bv_ref[...]
    vt_ref[...] = jnp.concatenate([vt, pad], axis=1)
    offs = lax.dot_general(wo_ref[...], qt, (((1,), (1,)), ((), ())),
                           preferred_element_type=jnp.float32) + bo_ref[...]
    p = j * BLK + lax.broadcasted_iota(jnp.int32, (1, BLK), 1)
    hpos = (p // W).astype(jnp.float32)
    wpos = (p % W).astype(jnp.float32)
    offs = offs.reshape(NREF, 2, BLK)
    ref_w = jnp.clip(wpos + offs[:, 0, :], 0.0, float(W - 1)).astype(jnp.int32)
    ref_h = jnp.clip(hpos + offs[:, 1, :], 0.0, float(H - 1)).astype(jnp.int32)
    idx_ref[...] = ref_h * W + ref_w + b * HW


def _stage1(x, Wq, bq, Wk, bk, Wv, bv, Wo, bo):
    xf = x.reshape(B, C, HW)
    grid = (B, N_BLK)
    wspec = pl.BlockSpec((C, C), lambda b, j: (0, 0))
    rspec = pl.BlockSpec((1, C), lambda b, j: (0, 0))
    return pl.pallas_call(
        _proj_body,
        grid=grid,
        in_specs=[
            pl.BlockSpec((1, C, BLK), lambda b, j: (b, 0, j)),
            wspec, rspec,
            wspec, rspec,
            wspec, rspec,
            pl.BlockSpec((2 * NREF, C), lambda b, j: (0, 0)),
            pl.BlockSpec((2 * NREF, 1), lambda b, j: (0, 0)),
        ],
        out_specs=[
            pl.BlockSpec((BLK, C), lambda b, j: (b * N_BLK + j, 0)),
            pl.BlockSpec((BLK, CP), lambda b, j: (b * N_BLK + j, 0)),
            pl.BlockSpec((BLK, CP), lambda b, j: (b * N_BLK + j, 0)),
            pl.BlockSpec((NREF, BLK), lambda b, j: (0, b * N_BLK + j)),
        ],
        out_shape=[
            jax.ShapeDtypeStruct((BHW, C), jnp.float32),
            jax.ShapeDtypeStruct((BHW, CP), jnp.float32),
            jax.ShapeDtypeStruct((BHW, CP), jnp.float32),
            jax.ShapeDtypeStruct((NREF, BHW), jnp.int32),
        ],
    )(xf, Wq, bq.reshape(1, C), Wk, bk.reshape(1, C), Wv, bv.reshape(1, C),
      Wo, bo.reshape(2 * NREF, 1))


NC, NS, L = 2, 16, 16
NW = NC * NS
P = 128
SUB = 8
NSUB = P // SUB
NCHUNK = BHW // P
NJ = C // L


def _sc_body(qt_hbm, kt_hbm, vt_hbm, idx_hbm, out_hbm,
             idxv, kg, vg, qv, ov, sem0, sem1):
    wid = lax.axis_index("s") * NC + lax.axis_index("c")
    nchunks = jnp.where(wid < NCHUNK % NW, NCHUNK // NW + 1, NCHUNK // NW)
    sems = (sem0, sem1)

    def chunk_body(t, _):
        ci = wid + t * NW
        base = pl.multiple_of(ci * P, P)
        pltpu.sync_copy(idx_hbm.at[:, pl.ds(base, P)], idxv)
        pltpu.sync_copy(qt_hbm.at[pl.ds(base, P)], qv)

        def fire(sub):
            slot = sub % 2
            cps = []
            for r in range(NREF):
                ixr = idxv.at[r, pl.ds(sub * SUB, SUB)]
                cps.append(pltpu.async_copy(kt_hbm.at[ixr], kg.at[slot, r],
                                            sems[slot]))
                cps.append(pltpu.async_copy(vt_hbm.at[ixr], vg.at[slot, r],
                                            sems[slot]))
            return cps

        pend = fire(0)
        for sub in range(NSUB):
            nxt = fire(sub + 1) if sub + 1 < NSUB else None
            for cp in pend:
                cp.wait()
            slot = sub % 2

            @plsc.parallel_loop(0, SUB, 1, unroll=2)
            def pos_body(i):
                pi = sub * SUB + i
                qvecs = []
                for jgrp in range(NJ):
                    qvecs.append(qv[pi, pl.ds(jgrp * L, L)])
                s = []
                for r in range(NREF):
                    acc = qvecs[0] * kg[slot, r, i, pl.ds(0, L)]
                    for jgrp in range(1, NJ):
                        acc = acc + qvecs[jgrp] * kg[slot, r, i, pl.ds(jgrp * L, L)]
                    s.append(jnp.sum(acc))
                for jgrp in range(NJ):
                    o = s[0] * vg[slot, 0, i, pl.ds(jgrp * L, L)]
                    for r in range(1, NREF):
                        o = o + s[r] * vg[slot, r, i, pl.ds(jgrp * L, L)]
                    ov[pi, pl.ds(jgrp * L, L)] = o

            pend = nxt

        pltpu.sync_copy(ov, out_hbm.at[pl.ds(base, P)])
        return ()

    lax.fori_loop(0, nchunks, chunk_body, (), unroll=False)


@functools.cache
def _sc_attn():
    return pl.kernel(
        _sc_body,
        out_type=jax.ShapeDtypeStruct((BHW, C), jnp.float32),
        mesh=plsc.VectorSubcoreMesh(core_axis_name="c", subcore_axis_name="s",
                                    num_cores=NC, num_subcores=NS),
        compiler_params=pltpu.CompilerParams(needs_layout_passes=False),
        scratch_types=[
            pltpu.VMEM((NREF, P), jnp.int32),
            pltpu.VMEM((2, NREF, SUB, CP), jnp.float32),
            pltpu.VMEM((2, NREF, SUB, CP), jnp.float32),
            pltpu.VMEM((P, C), jnp.float32),
            pltpu.VMEM((P, C), jnp.float32),
            pltpu.SemaphoreType.DMA,
            pltpu.SemaphoreType.DMA,
        ],
    )


TBLK = 512
NT_BLK = HW // TBLK


def _tr_body(ot_ref, o_ref):
    o_ref[0] = ot_ref[...].T


def _stage3(outT):
    return pl.pallas_call(
        _tr_body,
        grid=(B, NT_BLK),
        in_specs=[pl.BlockSpec((TBLK, C), lambda b, j: (b * NT_BLK + j, 0))],
        out_specs=pl.BlockSpec((1, C, TBLK), lambda b, j: (b, 0, j)),
        out_shape=jax.ShapeDtypeStruct((B, C, HW), jnp.float32),
    )(outT)


def kernel(x, Wq, bq, Wk, bk, Wv, bv, Wo, bo):
    qt, kt, vt, idx = _stage1(x, Wq, bq, Wk, bk, Wv, bv, Wo, bo)
    outT = _sc_attn()(qt, kt, vt, idx)
    return _stage3(outT).reshape(B, C, H, W)
